# Initial kernel scaffold; baseline (speedup 1.0000x reference)
import jax
import jax.numpy as jnp
from jax.experimental import pallas as pl


def kernel(x, edge_index, W_in, b_in, W_neigh0, W_self0, b_self0, W_neigh1, W_self1, b_self1, W_out, b_out):
    raise NotImplementedError("write your pallas kernel here")


# speedup vs baseline: 3.7604x; 3.7604x over previous
import functools

import jax
import jax.numpy as jnp
from jax import lax
from jax.experimental import pallas as pl
from jax.experimental.pallas import tpu as pltpu
from jax.experimental.pallas import tpu_sc as plsc

_K = 32
_BN = 1000
_CH = 80
_NCH = 250
_NS = 16
_RPS = 640
_SCN = 10240
_ACCR = 5120
_PASS = 5104
_BCH = 50


def _maxk(h):
    u = lax.bitcast_convert_type(h, jnp.uint32)
    neg = u >= jnp.uint32(0x80000000)
    key = jnp.where(neg, ~u, u | jnp.uint32(0x80000000))
    thresh = jnp.zeros((h.shape[0], 1), jnp.uint32)
    for bit in range(31, -1, -1):
        cand = thresh | jnp.uint32(1 << bit)
        cnt = jnp.sum((key >= cand).astype(jnp.int32), axis=1, keepdims=True)
        thresh = jnp.where(cnt >= _K, cand, thresh)
    return jnp.where(key >= thresh, h, jnp.zeros_like(h))


def _in_body(x_ref, wt_ref, b_ref, o0_ref, o1_ref):
    h = jnp.dot(x_ref[...], wt_ref[...], preferred_element_type=jnp.float32)
    h = _maxk(h + b_ref[...])
    o0_ref[...] = h[:, :128]
    o1_ref[...] = h[:, 128:]


def _tc_in(x, wt, b):
    n = x.shape[0]
    return pl.pallas_call(
        _in_body,
        grid=(n // _BN,),
        in_specs=[
            pl.BlockSpec((_BN, x.shape[1]), lambda i: (i, 0)),
            pl.BlockSpec(wt.shape, lambda i: (0, 0)),
            pl.BlockSpec(b.shape, lambda i: (0, 0)),
        ],
        out_specs=[pl.BlockSpec((_BN, 128), lambda i: (i, 0))] * 2,
        out_shape=[jax.ShapeDtypeStruct((n, 128), jnp.float32)] * 2,
    )(x, wt, b)


def _update(h0, h1, s0, s1, p0, p1, wst, wnt, b):
    cnt = p0[:, :1] + p1[:, :1]
    inv = 1.0 / jnp.maximum(cnt, 1.0)
    a0 = s0 * inv
    a1 = s1 * inv
    f32 = jnp.float32
    h = jnp.dot(h0, wst[:128], preferred_element_type=f32)
    h += jnp.dot(h1, wst[128:], preferred_element_type=f32)
    h += jnp.dot(a0, wnt[:128], preferred_element_type=f32)
    h += jnp.dot(a1, wnt[128:], preferred_element_type=f32)
    return h + b


def _mid_body(h0_ref, h1_ref, s0_ref, s1_ref, p0_ref, p1_ref, wst_ref, wnt_ref,
              b_ref, o0_ref, o1_ref):
    h = _update(h0_ref[...], h1_ref[...], s0_ref[...], s1_ref[...], p0_ref[...],
                p1_ref[...], wst_ref[...], wnt_ref[...], b_ref[...])
    h = _maxk(h)
    o0_ref[...] = h[:, :128]
    o1_ref[...] = h[:, 128:]


def _tc_mid(h0, h1, s0, s1, p0, p1, wst, wnt, b):
    n = h0.shape[0]
    row = lambda i: (i, 0)
    fix = lambda i: (0, 0)
    return pl.pallas_call(
        _mid_body,
        grid=(n // _BN,),
        in_specs=[
            pl.BlockSpec((_BN, 128), row), pl.BlockSpec((_BN, 128), row),
            pl.BlockSpec((_BN, 128), row), pl.BlockSpec((_BN, 128), row),
            pl.BlockSpec((_BN, 128), row), pl.BlockSpec((_BN, 128), row),
            pl.BlockSpec(wst.shape, fix), pl.BlockSpec(wnt.shape, fix),
            pl.BlockSpec(b.shape, fix),
        ],
        out_specs=[pl.BlockSpec((_BN, 128), row)] * 2,
        out_shape=[jax.ShapeDtypeStruct((n, 128), jnp.float32)] * 2,
    )(h0, h1, s0, s1, p0, p1, wst, wnt, b)


def _last_body(h0_ref, h1_ref, s0_ref, s1_ref, p0_ref, p1_ref, wst_ref, wnt_ref,
               b_ref, wot_ref, bo_ref, o_ref):
    h = _update(h0_ref[...], h1_ref[...], s0_ref[...], s1_ref[...], p0_ref[...],
                p1_ref[...], wst_ref[...], wnt_ref[...], b_ref[...])
    o_ref[...] = jnp.dot(h, wot_ref[...], preferred_element_type=jnp.float32) + bo_ref[...]


def _tc_last(h0, h1, s0, s1, p0, p1, wst, wnt, b, wot, bo):
    n = h0.shape[0]
    row = lambda i: (i, 0)
    fix = lambda i: (0, 0)
    return pl.pallas_call(
        _last_body,
        grid=(n // _BN,),
        in_specs=[
            pl.BlockSpec((_BN, 128), row), pl.BlockSpec((_BN, 128), row),
            pl.BlockSpec((_BN, 128), row), pl.BlockSpec((_BN, 128), row),
            pl.BlockSpec((_BN, 128), row), pl.BlockSpec((_BN, 128), row),
            pl.BlockSpec(wst.shape, fix), pl.BlockSpec(wnt.shape, fix),
            pl.BlockSpec(b.shape, fix), pl.BlockSpec(wot.shape, fix),
            pl.BlockSpec(bo.shape, fix),
        ],
        out_specs=pl.BlockSpec((_BN, wot.shape[1]), row),
        out_shape=jax.ShapeDtypeStruct((n, wot.shape[1]), jnp.float32),
    )(h0, h1, s0, s1, p0, p1, wst, wnt, b, wot, bo)


def _sc_agg(h0, h1, srcr, dstr):
    n = h0.shape[0]
    mesh = plsc.VectorSubcoreMesh(core_axis_name="c", subcore_axis_name="s")
    f32 = jnp.float32

    @functools.partial(
        pl.kernel,
        out_type=(jax.ShapeDtypeStruct((n, 128), f32),
                  jax.ShapeDtypeStruct((n, 128), f32)),
        mesh=mesh,
        scratch_types=[
            pltpu.VMEM((_BCH, _CH), jnp.int32),
            pltpu.VMEM((_BCH, _CH), jnp.int32),
            pltpu.VMEM((_CH, 128), f32),
            pltpu.VMEM((_CH, 128), f32),
            pltpu.VMEM((16, 128), f32),
            pltpu.VMEM_SHARED((_ACCR, 128), f32),
            pltpu.SemaphoreType.DMA,
            pltpu.SemaphoreType.DMA,
        ],
    )
    def agg(h0_hbm, h1_hbm, srcr_hbm, dstr_hbm, o0_hbm, o1_hbm,
            src_v, dst_v, rows0, rows1, zeros_v, acc, sem0, sem1):
        c = lax.axis_index("c")
        s = lax.axis_index("s")
        zv = jnp.zeros((16,), f32)
        for i in range(16):
            for j in range(8):
                zeros_v[i, pl.ds(j * 16, 16)] = zv

        def run(h_hbm, o_hbm, p):
            lo = jnp.int32(p * _PASS)

            def zero_body(i, _):
                pltpu.sync_copy(zeros_v, acc.at[pl.ds(s * 320 + i * 16, 16)])
                return 0
            lax.fori_loop(0, 20, zero_body, 0)
            plsc.subcore_barrier()

            def start(j, rows, sem):
                pltpu.make_async_copy(h_hbm.at[src_v.at[j]], rows, sem).start()

            def wait(j, rows, sem):
                pltpu.make_async_copy(h_hbm.at[src_v.at[j]], rows, sem).wait()

            def scat(j, rows):
                pltpu.sync_copy(rows, acc.at[dst_v.at[j]], add=True)

            def batch_body(b, _):
                pltpu.sync_copy(srcr_hbm.at[s].at[b], src_v)
                pltpu.sync_copy(dstr_hbm.at[s].at[b], dst_v)

                def remap_body(i, _):
                    for j in range(_CH // 16):
                        d = dst_v[i, pl.ds(j * 16, 16)]
                        inr = (d < _PASS) if p == 0 else (d >= _PASS)
                        trash = jnp.int32(_PASS) + (d & jnp.int32(15))
                        dst_v[i, pl.ds(j * 16, 16)] = jnp.where(inr, d - lo,
                                                                trash)
                    return 0
                lax.fori_loop(0, _BCH, remap_body, 0)

                start(0, rows0, sem0)

                def body(t, _):
                    j0 = 2 * t
                    wait(j0, rows0, sem0)
                    start(j0 + 1, rows1, sem1)
                    scat(j0, rows0)
                    wait(j0 + 1, rows1, sem1)
                    start(j0 + 2, rows0, sem0)
                    scat(j0 + 1, rows1)
                    return 0
                lax.fori_loop(0, _BCH // 2 - 1, body, 0)
                wait(_BCH - 2, rows0, sem0)
                start(_BCH - 1, rows1, sem1)
                scat(_BCH - 2, rows0)
                wait(_BCH - 1, rows1, sem1)
                scat(_BCH - 1, rows1)
                return 0
            lax.fori_loop(0, _NCH // _BCH, batch_body, 0)
            plsc.subcore_barrier()
            tail = 304 if p == 0 else 96

            @pl.when(s < _NS - 1)
            def _():
                pltpu.sync_copy(acc.at[pl.ds(s * 320, 320)],
                                o_hbm.at[pl.ds(p * _PASS + s * 320, 320)])

            @pl.when(s == _NS - 1)
            def _():
                pltpu.sync_copy(acc.at[pl.ds(4800, tail)],
                                o_hbm.at[pl.ds(p * _PASS + 4800, tail)])
            plsc.subcore_barrier()

        pl.when(c == 0)(lambda: run(h0_hbm, o0_hbm, 0))
        pl.when(c == 1)(lambda: run(h1_hbm, o1_hbm, 0))
        pl.when(c == 0)(lambda: run(h0_hbm, o0_hbm, 1))
        pl.when(c == 1)(lambda: run(h1_hbm, o1_hbm, 1))

    return agg(h0, h1, srcr, dstr)


def _sc_degree(dstr_deg, n):
    mesh = plsc.VectorSubcoreMesh(core_axis_name="c", subcore_axis_name="s")
    f32 = jnp.float32

    @functools.partial(
        pl.kernel,
        out_type=(jax.ShapeDtypeStruct((n, 128), f32),
                  jax.ShapeDtypeStruct((n, 128), f32)),
        mesh=mesh,
        scratch_types=[
            pltpu.VMEM((25, _CH), jnp.int32),
            pltpu.VMEM((_CH, 128), f32),
            pltpu.VMEM((16, 128), f32),
            pltpu.VMEM_SHARED((_ACCR, 128), f32),
        ],
    )
    def deg(dstr_hbm, p0_hbm, p1_hbm, idx_v, ones_v, zeros_v, acc):
        c = lax.axis_index("c")
        s = lax.axis_index("s")
        one = jnp.ones((16,), f32)
        zv = jnp.zeros((16,), f32)
        for i in range(_CH):
            for j in range(8):
                ones_v[i, pl.ds(j * 16, 16)] = one
        for i in range(16):
            for j in range(8):
                zeros_v[i, pl.ds(j * 16, 16)] = zv

        def run(o_hbm, p):
            lo = jnp.int32(p * _PASS)

            def zero_body(i, _):
                pltpu.sync_copy(zeros_v, acc.at[pl.ds(s * 320 + i * 16, 16)])
                return 0
            lax.fori_loop(0, 20, zero_body, 0)
            plsc.subcore_barrier()

            def batch_body(b, _):
                pltpu.sync_copy(dstr_hbm.at[c].at[s].at[b], idx_v)

                def remap_body(i, _):
                    for j in range(_CH // 16):
                        d = idx_v[i, pl.ds(j * 16, 16)]
                        inr = (d < _PASS) if p == 0 else (d >= _PASS)
                        trash = jnp.int32(_PASS) + (d & jnp.int32(15))
                        idx_v[i, pl.ds(j * 16, 16)] = jnp.where(inr, d - lo,
                                                                trash)
                    return 0
                lax.fori_loop(0, 25, remap_body, 0)

                def chunk_body(j, _):
                    pltpu.sync_copy(ones_v, acc.at[idx_v.at[j]], add=True)
                    return 0
                lax.fori_loop(0, 25, chunk_body, 0)
                return 0
            lax.fori_loop(0, 5, batch_body, 0)
            plsc.subcore_barrier()
            tail = 304 if p == 0 else 96

            @pl.when(s < _NS - 1)
            def _():
                pltpu.sync_copy(acc.at[pl.ds(s * 320, 320)],
                                o_hbm.at[pl.ds(p * _PASS + s * 320, 320)])

            @pl.when(s == _NS - 1)
            def _():
                pltpu.sync_copy(acc.at[pl.ds(4800, tail)],
                                o_hbm.at[pl.ds(p * _PASS + 4800, tail)])
            plsc.subcore_barrier()

        pl.when(c == 0)(lambda: run(p0_hbm, 0))
        pl.when(c == 1)(lambda: run(p1_hbm, 0))
        pl.when(c == 0)(lambda: run(p0_hbm, 1))
        pl.when(c == 1)(lambda: run(p1_hbm, 1))

    return deg(dstr_deg)


def kernel(x, edge_index, W_in, b_in, W_neigh0, W_self0, b_self0,
           W_neigh1, W_self1, b_self1, W_out, b_out):
    n = x.shape[0]
    e = edge_index.shape[1]
    assert e == _NS * _NCH * _CH and n % _BN == 0 and n <= _SCN == _NS * _RPS
    src = edge_index[0]
    dst = edge_index[1]
    srcr = src.reshape(_NS, _NCH // _BCH, _BCH, _CH)
    dstr = dst.reshape(_NS, _NCH // _BCH, _BCH, _CH)
    dstr_deg = dst.reshape(2, _NS, 5, 25, _CH)

    p0, p1 = _sc_degree(dstr_deg, n)
    h0, h1 = _tc_in(x, W_in.T, b_in.reshape(1, -1))
    s0, s1 = _sc_agg(h0, h1, srcr, dstr)
    h0, h1 = _tc_mid(h0, h1, s0, s1, p0, p1, W_self0.T, W_neigh0.T,
                     b_self0.reshape(1, -1))
    s0, s1 = _sc_agg(h0, h1, srcr, dstr)
    return _tc_last(h0, h1, s0, s1, p0, p1, W_self1.T, W_neigh1.T,
                    b_self1.reshape(1, -1), W_out.T, b_out.reshape(1, -1))
